# baseline (speedup 1.0000x reference)
import functools

import jax
import jax.numpy as jnp
from jax import lax
from jax.experimental import pallas as pl
from jax.experimental.pallas import tpu as pltpu
from jax.experimental.pallas import tpu_sc as plsc

NC, NS, L = 2, 16, 16
NW = NC * NS
ROWS, COLS = 128, 100000
RG = ROWS // L
RPW = ROWS // NW
T0 = 57344
CH = 168
CPW = 8
SPAN = CH * CPW
STRIDE = 1336
LAST0 = COLS - SPAN
CHT = 4096
NEG = float("-inf")


def _sc_body(pred_hbm, true_hbm, out_hbm, true_v, buf0, buf1, gbuf,
             part_v, sem0, sem1, semg):
    core = lax.axis_index("c")
    s = lax.axis_index("s")
    w = core * NS + s
    base = pl.multiple_of(
        jnp.where(w < NW - 1, T0 + w * STRIDE, LAST0), 8)

    pltpu.sync_copy(true_hbm, true_v)
    lane = lax.iota(jnp.int32, L)

    lane4 = lane & 3
    rowsel = RPW * w + lane4
    tsel = plsc.load_gather(true_v, [rowsel])
    gdesc = pltpu.async_copy(pred_hbm.at[tsel], gbuf, semg)

    bufs = (buf0, buf1)
    sems = (sem0, sem1)

    def dma(j, bslot):
        return pltpu.async_copy(
            pred_hbm.at[pl.ds(base + j * CH, CH), :], bufs[bslot],
            sems[bslot])

    accs = [jnp.full((L,), NEG, jnp.float32) for _ in range(RG)]
    tvs = [true_v[pl.ds(16 * j, L)] for j in range(RG)]

    descs = [None] * CPW
    descs[0] = dma(0, 0)
    for j in range(CPW):
        bslot = j % 2
        if j + 1 < CPW:
            descs[j + 1] = dma(j + 1, (j + 1) % 2)
        descs[j].wait()
        buf = bufs[bslot]
        c0 = base + j * CH

        for g in range(RG):
            p = tvs[g] - c0
            inr = (p >= 0) & (p < CH)
            pc = jnp.clip(p, 0, CH - 1)
            plsc.store_scatter(buf, [pc, lane + (16 * g)],
                               jnp.full((L,), NEG, jnp.float32), mask=inr)

        def class_body(i, a, _buf=buf):
            return tuple(
                jnp.maximum(a[k], _buf[i, pl.ds(16 * k, L)])
                for k in range(RG))

        accs = list(plsc.parallel_loop(0, CH, 1,
                                       carry=tuple(accs))(class_body))

    for g in range(RG):
        part_v[0, pl.ds(16 * g, L)] = accs[g]
        part_v[1, pl.ds(16 * g, L)] = jnp.full((L,), NEG, jnp.float32)

    gdesc.wait()
    dval = plsc.load_gather(gbuf, [lane4, rowsel])
    plsc.store_scatter(part_v, [jnp.broadcast_to(1, (L,)).astype(jnp.int32),
                                rowsel], dval, mask=lane < RPW)

    pltpu.sync_copy(part_v, out_hbm.at[w])


_sc_kernel = functools.partial(
    pl.kernel,
    out_type=jax.ShapeDtypeStruct((NW, 2, ROWS), jnp.float32),
    mesh=plsc.VectorSubcoreMesh(core_axis_name="c", subcore_axis_name="s",
                                num_cores=NC, num_subcores=NS),
    compiler_params=pltpu.CompilerParams(needs_layout_passes=False,
                                         skip_device_barrier=True),
    scratch_types=[
        pltpu.VMEM((ROWS,), jnp.int32),
        pltpu.VMEM((CH, ROWS), jnp.float32),
        pltpu.VMEM((CH, ROWS), jnp.float32),
        pltpu.VMEM((L, ROWS), jnp.float32),
        pltpu.VMEM((2, ROWS), jnp.float32),
        pltpu.SemaphoreType.DMA,
        pltpu.SemaphoreType.DMA,
        pltpu.SemaphoreType.DMA,
    ],
)(_sc_body)


def _tc_body(x_ref, m_ref):
    i = pl.program_id(0)

    @pl.when(i == 0)
    def _():
        m_ref[...] = jnp.full((16, ROWS), NEG, jnp.float32)

    x = x_ref[...]
    a, b = x[0:CHT // 2], x[CHT // 2:CHT]
    m1, m2 = jnp.maximum(a, b), jnp.minimum(a, b)
    n = CHT // 2
    while n > 8:
        h = n // 2
        a1, b1 = m1[:h], m1[h:]
        a2, b2 = m2[:h], m2[h:]
        m1, m2 = (jnp.maximum(a1, b1),
                  jnp.maximum(jnp.minimum(a1, b1), jnp.maximum(a2, b2)))
        n = h
    r1, r2 = m_ref[0:8, :], m_ref[8:16, :]
    m_ref[0:8, :] = jnp.maximum(r1, m1)
    m_ref[8:16, :] = jnp.maximum(jnp.minimum(r1, m1),
                                 jnp.maximum(r2, m2))


def _tc_head(predT):
    return pl.pallas_call(
        _tc_body,
        grid=(T0 // CHT,),
        in_specs=[pl.BlockSpec((CHT, ROWS), lambda i: (i, 0))],
        out_specs=pl.BlockSpec((16, ROWS), lambda i: (0, 0)),
        out_shape=jax.ShapeDtypeStruct((16, ROWS), jnp.float32),
    )(predT)


def _fin_body(x_ref, mm_ref, true_ref, o_ref):
    m_sc = jnp.max(x_ref[:, 0, :], axis=0)
    t = jnp.max(x_ref[:, 1, :], axis=0)
    a1, a2 = mm_ref[0:1, :], mm_ref[8:9, :]
    for k in range(1, 8):
        b1, b2 = mm_ref[k:k + 1, :], mm_ref[8 + k:9 + k, :]
        a1, a2 = (jnp.maximum(a1, b1),
                  jnp.maximum(jnp.minimum(a1, b1), jnp.maximum(a2, b2)))
    m1, m2 = a1[0], a2[0]
    cond = (true_ref[0] < T0) & (t == m1)
    m = jnp.maximum(m_sc, jnp.where(cond, m2, m1))
    o_ref[...] = jnp.sum(m - t).reshape(1, 1)


def _finish(partials, tc_mm, true2d):
    return pl.pallas_call(
        _fin_body,
        out_shape=jax.ShapeDtypeStruct((1, 1), jnp.float32),
    )(partials, tc_mm, true2d)


@jax.jit
def kernel(pred, true):
    true32 = true.astype(jnp.int32)
    predT = pred.T
    tc_mm = _tc_head(predT)
    partials = _sc_kernel(predT, true32)
    return _finish(partials, tc_mm, true32.reshape(1, ROWS))[0, 0]


# speedup vs baseline: 1.0011x; 1.0011x over previous
import functools

import jax
import jax.numpy as jnp
from jax import lax
from jax.experimental import pallas as pl
from jax.experimental.pallas import tpu as pltpu
from jax.experimental.pallas import tpu_sc as plsc

NC, NS, L = 2, 16, 16
NW = NC * NS
ROWS, COLS = 128, 100000
RG = ROWS // L
RPW = ROWS // NW
T0 = 57344
CH = 168
CPW = 8
SPAN = CH * CPW
STRIDE = 1336
LAST0 = COLS - SPAN
CHT = 8192
NEG = float("-inf")


def _sc_body(pred_hbm, true_hbm, out_hbm, true_v, buf0, buf1, gbuf,
             part_v, sem0, sem1, semg):
    core = lax.axis_index("c")
    s = lax.axis_index("s")
    w = core * NS + s
    base = pl.multiple_of(
        jnp.where(w < NW - 1, T0 + w * STRIDE, LAST0), 8)

    pltpu.sync_copy(true_hbm, true_v)
    lane = lax.iota(jnp.int32, L)

    lane4 = lane & 3
    rowsel = RPW * w + lane4
    tsel = plsc.load_gather(true_v, [rowsel])
    gdesc = pltpu.async_copy(pred_hbm.at[tsel], gbuf, semg)

    bufs = (buf0, buf1)
    sems = (sem0, sem1)

    def dma(j, bslot):
        return pltpu.async_copy(
            pred_hbm.at[pl.ds(base + j * CH, CH), :], bufs[bslot],
            sems[bslot])

    accs = [jnp.full((L,), NEG, jnp.float32) for _ in range(RG)]
    tvs = [true_v[pl.ds(16 * j, L)] for j in range(RG)]

    descs = [None] * CPW
    descs[0] = dma(0, 0)
    for j in range(CPW):
        bslot = j % 2
        if j + 1 < CPW:
            descs[j + 1] = dma(j + 1, (j + 1) % 2)
        descs[j].wait()
        buf = bufs[bslot]
        c0 = base + j * CH

        for g in range(RG):
            p = tvs[g] - c0
            inr = (p >= 0) & (p < CH)
            pc = jnp.clip(p, 0, CH - 1)
            plsc.store_scatter(buf, [pc, lane + (16 * g)],
                               jnp.full((L,), NEG, jnp.float32), mask=inr)

        def class_body(i, a, _buf=buf):
            return tuple(
                jnp.maximum(a[k], _buf[i, pl.ds(16 * k, L)])
                for k in range(RG))

        accs = list(plsc.parallel_loop(0, CH, 1,
                                       carry=tuple(accs))(class_body))

    for g in range(RG):
        part_v[0, pl.ds(16 * g, L)] = accs[g]
        part_v[1, pl.ds(16 * g, L)] = jnp.full((L,), NEG, jnp.float32)

    gdesc.wait()
    dval = plsc.load_gather(gbuf, [lane4, rowsel])
    plsc.store_scatter(part_v, [jnp.broadcast_to(1, (L,)).astype(jnp.int32),
                                rowsel], dval, mask=lane < RPW)

    pltpu.sync_copy(part_v, out_hbm.at[w])


_sc_kernel = functools.partial(
    pl.kernel,
    out_type=jax.ShapeDtypeStruct((NW, 2, ROWS), jnp.float32),
    mesh=plsc.VectorSubcoreMesh(core_axis_name="c", subcore_axis_name="s",
                                num_cores=NC, num_subcores=NS),
    compiler_params=pltpu.CompilerParams(needs_layout_passes=False),
    scratch_types=[
        pltpu.VMEM((ROWS,), jnp.int32),
        pltpu.VMEM((CH, ROWS), jnp.float32),
        pltpu.VMEM((CH, ROWS), jnp.float32),
        pltpu.VMEM((L, ROWS), jnp.float32),
        pltpu.VMEM((2, ROWS), jnp.float32),
        pltpu.SemaphoreType.DMA,
        pltpu.SemaphoreType.DMA,
        pltpu.SemaphoreType.DMA,
    ],
)(_sc_body)


def _tc_body(x_ref, m_ref):
    i = pl.program_id(0)

    @pl.when(i == 0)
    def _():
        m_ref[...] = jnp.full((16, ROWS), NEG, jnp.float32)

    x = x_ref[...]
    a, b = x[0:CHT // 2], x[CHT // 2:CHT]
    m1, m2 = jnp.maximum(a, b), jnp.minimum(a, b)
    n = CHT // 2
    while n > 8:
        h = n // 2
        a1, b1 = m1[:h], m1[h:]
        a2, b2 = m2[:h], m2[h:]
        m1, m2 = (jnp.maximum(a1, b1),
                  jnp.maximum(jnp.minimum(a1, b1), jnp.maximum(a2, b2)))
        n = h
    r1, r2 = m_ref[0:8, :], m_ref[8:16, :]
    m_ref[0:8, :] = jnp.maximum(r1, m1)
    m_ref[8:16, :] = jnp.maximum(jnp.minimum(r1, m1),
                                 jnp.maximum(r2, m2))


def _tc_head(predT):
    return pl.pallas_call(
        _tc_body,
        grid=(T0 // CHT,),
        in_specs=[pl.BlockSpec((CHT, ROWS), lambda i: (i, 0))],
        out_specs=pl.BlockSpec((16, ROWS), lambda i: (0, 0)),
        out_shape=jax.ShapeDtypeStruct((16, ROWS), jnp.float32),
    )(predT)


def _fin_body(x_ref, mm_ref, true_ref, o_ref):
    m_sc = jnp.max(x_ref[:, 0, :], axis=0)
    t = jnp.max(x_ref[:, 1, :], axis=0)
    a1, a2 = mm_ref[0:1, :], mm_ref[8:9, :]
    for k in range(1, 8):
        b1, b2 = mm_ref[k:k + 1, :], mm_ref[8 + k:9 + k, :]
        a1, a2 = (jnp.maximum(a1, b1),
                  jnp.maximum(jnp.minimum(a1, b1), jnp.maximum(a2, b2)))
    m1, m2 = a1[0], a2[0]
    cond = (true_ref[0] < T0) & (t == m1)
    m = jnp.maximum(m_sc, jnp.where(cond, m2, m1))
    o_ref[...] = jnp.sum(m - t).reshape(1, 1)


def _finish(partials, tc_mm, true2d):
    return pl.pallas_call(
        _fin_body,
        out_shape=jax.ShapeDtypeStruct((1, 1), jnp.float32),
    )(partials, tc_mm, true2d)


@jax.jit
def kernel(pred, true):
    true32 = true.astype(jnp.int32)
    predT = pred.T
    tc_mm = _tc_head(predT)
    partials = _sc_kernel(predT, true32)
    return _finish(partials, tc_mm, true32.reshape(1, ROWS))[0, 0]


# speedup vs baseline: 1.0220x; 1.0208x over previous
import functools

import jax
import jax.numpy as jnp
from jax import lax
from jax.experimental import pallas as pl
from jax.experimental.pallas import tpu as pltpu
from jax.experimental.pallas import tpu_sc as plsc

NC, NS, L = 2, 16, 16
NW = NC * NS
ROWS, COLS = 128, 100000
RG = ROWS // L
RPW = ROWS // NW
T0 = 57344
CH = 168
CPW = 8
SPAN = CH * CPW
STRIDE = 1336
LAST0 = COLS - SPAN
CHT = 4096
NEG = float("-inf")


def _sc_body(pred_hbm, true_hbm, out_hbm, true_v, buf0, buf1, gbuf,
             part_v, sem0, sem1, semg):
    core = lax.axis_index("c")
    s = lax.axis_index("s")
    w = core * NS + s
    base = pl.multiple_of(
        jnp.where(w < NW - 1, T0 + w * STRIDE, LAST0), 8)

    pltpu.sync_copy(true_hbm, true_v)
    lane = lax.iota(jnp.int32, L)

    lane4 = lane & 3
    rowsel = RPW * w + lane4
    tsel = plsc.load_gather(true_v, [rowsel])
    gdesc = pltpu.async_copy(pred_hbm.at[tsel], gbuf, semg)

    bufs = (buf0, buf1)
    sems = (sem0, sem1)

    def src(j):
        return pred_hbm.at[pl.ds(pl.multiple_of(base + j * CH, 8), CH), :]

    tvs = [true_v[pl.ds(16 * j, L)] for j in range(RG)]

    pltpu.async_copy(src(0), buf0, sem0)
    pltpu.async_copy(src(1), buf1, sem1)

    def pair_body(t, accs):
        j0 = 2 * t
        for b, (buf, sem) in enumerate(((buf0, sem0), (buf1, sem1))):
            j = j0 + b
            pltpu.make_async_copy(src(j), buf, sem).wait()

            @pl.when(j + 2 < CPW)
            def _():
                pltpu.async_copy(src(j + 2), buf, sem)

            c0 = base + j * CH
            for g in range(RG):
                p = tvs[g] - c0
                inr = (p >= 0) & (p < CH)
                pc = jnp.clip(p, 0, CH - 1)
                plsc.store_scatter(buf, [pc, lane + (16 * g)],
                                   jnp.full((L,), NEG, jnp.float32),
                                   mask=inr)

            def class_body(i, a, _buf=buf):
                return tuple(
                    jnp.maximum(a[k], _buf[i, pl.ds(16 * k, L)])
                    for k in range(RG))

            accs = plsc.parallel_loop(0, CH, 1, carry=accs)(class_body)
        return accs

    accs = lax.fori_loop(
        0, CPW // 2, pair_body,
        tuple(jnp.full((L,), NEG, jnp.float32) for _ in range(RG)))

    for g in range(RG):
        part_v[0, pl.ds(16 * g, L)] = accs[g]
        part_v[1, pl.ds(16 * g, L)] = jnp.full((L,), NEG, jnp.float32)

    gdesc.wait()
    dval = plsc.load_gather(gbuf, [lane4, rowsel])
    plsc.store_scatter(part_v, [jnp.broadcast_to(1, (L,)).astype(jnp.int32),
                                rowsel], dval, mask=lane < RPW)

    pltpu.sync_copy(part_v, out_hbm.at[w])


_sc_kernel = functools.partial(
    pl.kernel,
    out_type=jax.ShapeDtypeStruct((NW, 2, ROWS), jnp.float32),
    mesh=plsc.VectorSubcoreMesh(core_axis_name="c", subcore_axis_name="s",
                                num_cores=NC, num_subcores=NS),
    compiler_params=pltpu.CompilerParams(needs_layout_passes=False),
    scratch_types=[
        pltpu.VMEM((ROWS,), jnp.int32),
        pltpu.VMEM((CH, ROWS), jnp.float32),
        pltpu.VMEM((CH, ROWS), jnp.float32),
        pltpu.VMEM((L, ROWS), jnp.float32),
        pltpu.VMEM((2, ROWS), jnp.float32),
        pltpu.SemaphoreType.DMA,
        pltpu.SemaphoreType.DMA,
        pltpu.SemaphoreType.DMA,
    ],
)(_sc_body)


def _tc_body(x_ref, m_ref):
    i = pl.program_id(0)

    @pl.when(i == 0)
    def _():
        m_ref[...] = jnp.full((16, ROWS), NEG, jnp.float32)

    x = x_ref[...]
    a, b = x[0:CHT // 2], x[CHT // 2:CHT]
    m1, m2 = jnp.maximum(a, b), jnp.minimum(a, b)
    n = CHT // 2
    while n > 8:
        h = n // 2
        a1, b1 = m1[:h], m1[h:]
        a2, b2 = m2[:h], m2[h:]
        m1, m2 = (jnp.maximum(a1, b1),
                  jnp.maximum(jnp.minimum(a1, b1), jnp.maximum(a2, b2)))
        n = h
    r1, r2 = m_ref[0:8, :], m_ref[8:16, :]
    m_ref[0:8, :] = jnp.maximum(r1, m1)
    m_ref[8:16, :] = jnp.maximum(jnp.minimum(r1, m1),
                                 jnp.maximum(r2, m2))


def _tc_head(predT):
    return pl.pallas_call(
        _tc_body,
        grid=(T0 // CHT,),
        in_specs=[pl.BlockSpec((CHT, ROWS), lambda i: (i, 0))],
        out_specs=pl.BlockSpec((16, ROWS), lambda i: (0, 0)),
        out_shape=jax.ShapeDtypeStruct((16, ROWS), jnp.float32),
    )(predT)


def _fin_body(x_ref, mm_ref, true_ref, o_ref):
    m_sc = jnp.max(x_ref[:, 0, :], axis=0)
    t = jnp.max(x_ref[:, 1, :], axis=0)
    a1, a2 = mm_ref[0:1, :], mm_ref[8:9, :]
    for k in range(1, 8):
        b1, b2 = mm_ref[k:k + 1, :], mm_ref[8 + k:9 + k, :]
        a1, a2 = (jnp.maximum(a1, b1),
                  jnp.maximum(jnp.minimum(a1, b1), jnp.maximum(a2, b2)))
    m1, m2 = a1[0], a2[0]
    cond = (true_ref[0] < T0) & (t == m1)
    m = jnp.maximum(m_sc, jnp.where(cond, m2, m1))
    o_ref[...] = jnp.sum(m - t).reshape(1, 1)


def _finish(partials, tc_mm, true2d):
    return pl.pallas_call(
        _fin_body,
        out_shape=jax.ShapeDtypeStruct((1, 1), jnp.float32),
    )(partials, tc_mm, true2d)


@jax.jit
def kernel(pred, true):
    true32 = true.astype(jnp.int32)
    predT = pred.T
    tc_mm = _tc_head(predT)
    partials = _sc_kernel(predT, true32)
    return _finish(partials, tc_mm, true32.reshape(1, ROWS))[0, 0]
